# baseline (speedup 1.0000x reference)
import functools

import jax
import jax.numpy as jnp
from jax import lax
from jax.experimental import pallas as pl
from jax.experimental.pallas import tpu as pltpu
from jax.experimental.pallas import tpu_sc as plsc

N_NODES = 10000
E = 320000
D = 128
HID = 16
NREL = 5

NC, NS = 2, 16
NW = NC * NS
EPW = E // NW
CH = 80
NCHUNK = EPW // CH
NBUF = 5
NJ = NCHUNK // NBUF

BT = 8000
NBT = E // BT

HIMASK = -65536


def _sc_gather_mul(x, row, col):
    mesh = plsc.VectorSubcoreMesh(
        core_axis_name="c", subcore_axis_name="s", num_cores=NC, num_subcores=NS)

    @functools.partial(
        pl.kernel,
        out_type=jax.ShapeDtypeStruct((E // 2, D), jnp.int32),
        mesh=mesh,
        compiler_params=pltpu.CompilerParams(
            needs_layout_passes=False, use_tc_tiling_on_sc=False),
        scratch_types=[
            pltpu.VMEM((EPW,), jnp.int32),
            pltpu.VMEM((EPW,), jnp.int32),
            [pltpu.VMEM((CH, D // 2), jnp.int32) for _ in range(NBUF)],
            [pltpu.VMEM((CH, D // 2), jnp.int32) for _ in range(NBUF)],
            [pltpu.VMEM((CH // 2, D), jnp.int32) for _ in range(NBUF)],
            [pltpu.SemaphoreType.DMA for _ in range(NBUF)],
            [pltpu.SemaphoreType.DMA for _ in range(NBUF)],
            [pltpu.SemaphoreType.DMA for _ in range(NBUF)],
        ],
    )
    def k(x_hbm, row_hbm, col_hbm, y_hbm, idxr, idxc, xi, xj, yb, smr, smc,
          sst):
        wid = lax.axis_index("s") * NC + lax.axis_index("c")
        base = pl.multiple_of(wid * EPW, EPW)
        pltpu.sync_copy(row_hbm.at[pl.ds(base, EPW)], idxr)
        pltpu.sync_copy(col_hbm.at[pl.ds(base, EPW)], idxc)

        def fire(ci, b):
            off = pl.multiple_of(ci * CH, CH)
            pltpu.async_copy(x_hbm.at[idxr.at[pl.ds(off, CH)]], xi[b], smr[b])
            pltpu.async_copy(x_hbm.at[idxc.at[pl.ds(off, CH)]], xj[b], smc[b])

        def gwait(b):
            pltpu.make_async_copy(x_hbm.at[idxr.at[pl.ds(0, CH)]], xi[b],
                                  smr[b]).wait()
            pltpu.make_async_copy(x_hbm.at[idxc.at[pl.ds(0, CH)]], xj[b],
                                  smc[b]).wait()

        def swait(b):
            pltpu.make_async_copy(yb[b], y_hbm.at[pl.ds(0, CH // 2)],
                                  sst[b]).wait()

        for b in range(NBUF - 1):
            fire(b, b)

        def outer(j, carry):
            for b in range(NBUF):
                ci = j * NBUF + b
                gwait(b)

                def rowmul(r2, c2):
                    for half in range(2):
                        r = r2 * 2 + half
                        for kk in range(D // 32):
                            s = pl.ds(kk * 16, 16)
                            vi = xi[b][r, s]
                            vj = xj[b][r, s]
                            a0 = plsc.bitcast(vi << 16, jnp.float32)
                            b0 = plsc.bitcast(vj << 16, jnp.float32)
                            a1 = plsc.bitcast(vi & HIMASK, jnp.float32)
                            b1 = plsc.bitcast(vj & HIMASK, jnp.float32)
                            p0 = plsc.bitcast(a0 * b0, jnp.int32)
                            p1 = plsc.bitcast(a1 * b1, jnp.int32)
                            q0 = lax.shift_right_logical(p0 + 0x8000, 16)
                            q1 = (p1 + 0x8000) & HIMASK
                            yb[b][r2, pl.ds(half * 64 + kk * 16, 16)] = (
                                q0 | q1)
                    return c2

                lax.fori_loop(0, CH // 2, rowmul, 0, unroll=2)
                off = pl.multiple_of(ci * CH, CH)
                pltpu.async_copy(yb[b],
                                 y_hbm.at[pl.ds((base + off) // 2, CH // 2)],
                                 sst[b])
                nb = (b + NBUF - 1) % NBUF
                if b == 0:
                    @pl.when(j > 0)
                    def _():
                        swait(nb)
                    fire(ci + NBUF - 1, nb)
                else:
                    @pl.when(j < NJ - 1)
                    def _():
                        swait(nb)
                        fire(ci + NBUF - 1, nb)
            return carry

        lax.fori_loop(0, NJ, outer, 0)
        for b in range(NBUF):
            swait(b)

    return k(x, row, col)


def _tc_mlp(y, concs, rel2d, W0a, wc, b0r, Wf, bf):
    def body(y_ref, concs_ref, rel_ref, W0a_ref, wc_ref, b0_ref, Wf_ref,
             bf_ref, ones_ref, o_ref):
        yb = y_ref[...]
        cc = concs_ref[...]
        c = cc[:, 0:1] * cc[:, 1:2]
        h = jnp.dot(yb, W0a_ref[...], preferred_element_type=jnp.float32)
        h = h + c * wc_ref[...] + b0_ref[...]
        h = jnp.maximum(h, 0.0)
        o5 = jnp.dot(h, Wf_ref[...], preferred_element_type=jnp.float32)
        o5 = o5 + bf_ref[...]
        rel = rel_ref[...]
        onehot = (rel == lax.broadcasted_iota(jnp.int32, (1, NREL), 1))
        sel = o5 * onehot.astype(jnp.float32)
        o_ref[...] = jnp.dot(sel, ones_ref[...],
                             preferred_element_type=jnp.float32)

    ones5 = jnp.ones((NREL, 1), jnp.float32)
    return pl.pallas_call(
        body,
        grid=(NBT,),
        in_specs=[
            pl.BlockSpec((BT, D), lambda i: (i, 0)),
            pl.BlockSpec((BT, 2), lambda i: (i, 0)),
            pl.BlockSpec((BT, 1), lambda i: (i, 0)),
            pl.BlockSpec((D, HID), lambda i: (0, 0)),
            pl.BlockSpec((1, HID), lambda i: (0, 0)),
            pl.BlockSpec((1, HID), lambda i: (0, 0)),
            pl.BlockSpec((HID, NREL), lambda i: (0, 0)),
            pl.BlockSpec((1, NREL), lambda i: (0, 0)),
            pl.BlockSpec((NREL, 1), lambda i: (0, 0)),
        ],
        out_specs=pl.BlockSpec((BT, 1), lambda i: (i, 0)),
        out_shape=jax.ShapeDtypeStruct((E, 1), jnp.float32),
    )(y, concs, rel2d, W0a, wc, b0r, Wf, bf, ones5)


def kernel(x, edge_index, relations, concs, W0, b0, Wr, br):
    row = edge_index[:, 0]
    col = edge_index[:, 1]
    x32 = lax.bitcast_convert_type(
        x.astype(jnp.bfloat16).reshape(N_NODES, D // 2, 2), jnp.int32)
    y32 = _sc_gather_mul(x32, row, col)
    y = lax.bitcast_convert_type(y32, jnp.bfloat16).reshape(E, D)
    W0a = W0[:D].astype(jnp.bfloat16)
    wc = W0[D:D + 1, :]
    Wf = Wr[:, :, 0].T
    bf = br[:, 0][None, :]
    return _tc_mlp(y, concs, relations[:, None], W0a, wc, b0[None, :], Wf, bf)


# speedup vs baseline: 27.3009x; 27.3009x over previous
import functools

import jax
import jax.numpy as jnp
from jax import lax
from jax.experimental import pallas as pl
from jax.experimental.pallas import tpu as pltpu
from jax.experimental.pallas import tpu_sc as plsc

N_NODES = 10000
E = 320000
D = 128
HID = 16
NREL = 5

NC, NS = 2, 16
NW = NC * NS
EPW = E // NW
CH = 80
NCHUNK = EPW // CH
NBUF = 5
NJ = NCHUNK // NBUF

BT2 = 4000
NBT = E // 2 // BT2

HIMASK = -65536


def _sc_gather_mul(x, row, col):
    mesh = plsc.VectorSubcoreMesh(
        core_axis_name="c", subcore_axis_name="s", num_cores=NC, num_subcores=NS)

    @functools.partial(
        pl.kernel,
        out_type=jax.ShapeDtypeStruct((E // 2, D), jnp.int32),
        mesh=mesh,
        compiler_params=pltpu.CompilerParams(
            needs_layout_passes=False, use_tc_tiling_on_sc=False),
        scratch_types=[
            pltpu.VMEM((EPW,), jnp.int32),
            pltpu.VMEM((EPW,), jnp.int32),
            [pltpu.VMEM((CH, D // 2), jnp.int32) for _ in range(NBUF)],
            [pltpu.VMEM((CH, D // 2), jnp.int32) for _ in range(NBUF)],
            [pltpu.VMEM((CH // 2, D), jnp.int32) for _ in range(NBUF)],
            [pltpu.SemaphoreType.DMA for _ in range(NBUF)],
            [pltpu.SemaphoreType.DMA for _ in range(NBUF)],
            [pltpu.SemaphoreType.DMA for _ in range(NBUF)],
        ],
    )
    def k(x_hbm, row_hbm, col_hbm, y_hbm, idxr, idxc, xi, xj, yb, smr, smc,
          sst):
        wid = lax.axis_index("s") * NC + lax.axis_index("c")
        base = pl.multiple_of(wid * EPW, EPW)
        pltpu.sync_copy(row_hbm.at[pl.ds(base, EPW)], idxr)
        pltpu.sync_copy(col_hbm.at[pl.ds(base, EPW)], idxc)

        def fire(ci, b):
            off = pl.multiple_of(ci * CH, CH)
            pltpu.async_copy(x_hbm.at[idxr.at[pl.ds(off, CH)]], xi[b], smr[b])
            pltpu.async_copy(x_hbm.at[idxc.at[pl.ds(off, CH)]], xj[b], smc[b])

        def gwait(b):
            pltpu.make_async_copy(x_hbm.at[idxr.at[pl.ds(0, CH)]], xi[b],
                                  smr[b]).wait()
            pltpu.make_async_copy(x_hbm.at[idxc.at[pl.ds(0, CH)]], xj[b],
                                  smc[b]).wait()

        def swait(b):
            pltpu.make_async_copy(yb[b], y_hbm.at[pl.ds(0, CH // 2)],
                                  sst[b]).wait()

        for b in range(NBUF - 1):
            fire(b, b)

        def outer(j, carry):
            for b in range(NBUF):
                ci = j * NBUF + b
                gwait(b)

                def rowmul(r2, c2):
                    for half in range(2):
                        r = r2 * 2 + half
                        for kk in range(D // 32):
                            s = pl.ds(kk * 16, 16)
                            vi = xi[b][r, s]
                            vj = xj[b][r, s]
                            a0 = plsc.bitcast(vi << 16, jnp.float32)
                            b0 = plsc.bitcast(vj << 16, jnp.float32)
                            a1 = plsc.bitcast(vi & HIMASK, jnp.float32)
                            b1 = plsc.bitcast(vj & HIMASK, jnp.float32)
                            p0 = plsc.bitcast(a0 * b0, jnp.int32)
                            p1 = plsc.bitcast(a1 * b1, jnp.int32)
                            q0 = lax.shift_right_logical(p0 + 0x8000, 16)
                            q1 = (p1 + 0x8000) & HIMASK
                            yb[b][r2, pl.ds(half * 64 + kk * 16, 16)] = (
                                q0 | q1)
                    return c2

                lax.fori_loop(0, CH // 2, rowmul, 0, unroll=2)
                off = pl.multiple_of(ci * CH, CH)
                pltpu.async_copy(yb[b],
                                 y_hbm.at[pl.ds((base + off) // 2, CH // 2)],
                                 sst[b])
                nb = (b + NBUF - 1) % NBUF
                if b == 0:
                    @pl.when(j > 0)
                    def _():
                        swait(nb)
                    fire(ci + NBUF - 1, nb)
                else:
                    @pl.when(j < NJ - 1)
                    def _():
                        swait(nb)
                        fire(ci + NBUF - 1, nb)
            return carry

        lax.fori_loop(0, NJ, outer, 0)
        for b in range(NBUF):
            swait(b)

    return k(x, row, col)


def _tc_mlp(y32, c4, rel2, W0a, wc, b0r, Wf, bf):
    def body(y_ref, c4_ref, rel_ref, W0a_ref, wc_ref, b0_ref, Wf_ref,
             bf_ref, ones_ref, o_ref):
        v = y_ref[...]
        ylo = lax.bitcast_convert_type(v << 16, jnp.float32)
        yhi = lax.bitcast_convert_type(v & HIMASK, jnp.float32)
        ylo = ylo.astype(jnp.bfloat16)
        yhi = yhi.astype(jnp.bfloat16)
        cc = c4_ref[...]
        rel = rel_ref[...]
        outs = []
        for half in range(2):
            sl = slice(half * 64, half * 64 + 64)
            yb = jnp.concatenate([ylo[:, sl], yhi[:, sl]], axis=1)
            c = cc[:, 2 * half:2 * half + 1] * cc[:, 2 * half + 1:2 * half + 2]
            h = jnp.dot(yb, W0a_ref[...], preferred_element_type=jnp.float32)
            h = jnp.maximum(h + c * wc_ref[...] + b0_ref[...], 0.0)
            o5 = jnp.dot(h, Wf_ref[...], preferred_element_type=jnp.float32)
            o5 = o5 + bf_ref[...]
            onehot = (rel[:, half:half + 1]
                      == lax.broadcasted_iota(jnp.int32, (1, NREL), 1))
            sel = o5 * onehot.astype(jnp.float32)
            outs.append(jnp.dot(sel, ones_ref[...],
                                preferred_element_type=jnp.float32))
        o_ref[...] = jnp.concatenate(outs, axis=1)

    ones5 = jnp.ones((NREL, 1), jnp.float32)
    return pl.pallas_call(
        body,
        grid=(NBT,),
        in_specs=[
            pl.BlockSpec((BT2, D), lambda i: (i, 0)),
            pl.BlockSpec((BT2, 4), lambda i: (i, 0)),
            pl.BlockSpec((BT2, 2), lambda i: (i, 0)),
            pl.BlockSpec((D, HID), lambda i: (0, 0)),
            pl.BlockSpec((1, HID), lambda i: (0, 0)),
            pl.BlockSpec((1, HID), lambda i: (0, 0)),
            pl.BlockSpec((HID, NREL), lambda i: (0, 0)),
            pl.BlockSpec((1, NREL), lambda i: (0, 0)),
            pl.BlockSpec((NREL, 1), lambda i: (0, 0)),
        ],
        out_specs=pl.BlockSpec((BT2, 2), lambda i: (i, 0)),
        out_shape=jax.ShapeDtypeStruct((E // 2, 2), jnp.float32),
    )(y32, c4, rel2, W0a, wc, b0r, Wf, bf, ones5)


def kernel(x, edge_index, relations, concs, W0, b0, Wr, br):
    row = edge_index[:, 0]
    col = edge_index[:, 1]
    x32 = lax.bitcast_convert_type(
        x.astype(jnp.bfloat16).reshape(N_NODES, D // 2, 2), jnp.int32)
    y32 = _sc_gather_mul(x32, row, col)
    W0ab = W0[:D].astype(jnp.bfloat16)
    W0a = jnp.concatenate([W0ab[0::2], W0ab[1::2]], axis=0)
    wc = W0[D:D + 1, :]
    Wf = Wr[:, :, 0].T
    bf = br[:, 0][None, :]
    out2 = _tc_mlp(y32, concs.reshape(E // 2, 4), relations.reshape(E // 2, 2),
                   W0a, wc, b0[None, :], Wf, bf)
    return out2.reshape(E, 1)
